# baseline (speedup 1.0000x reference)
import functools

import jax
import jax.numpy as jnp
from jax import lax
from jax.experimental import pallas as pl
from jax.experimental.pallas import tpu as pltpu
from jax.experimental.pallas import tpu_sc as plsc

_B, _S, _D = 4096, 26, 64
_NW = 32
_CH = 128
_NSLAB = _S


def _gather(xk, table):
    mesh = plsc.VectorSubcoreMesh(core_axis_name="c", subcore_axis_name="s")

    @functools.partial(
        pl.kernel,
        mesh=mesh,
        out_type=jax.ShapeDtypeStruct((_S, 8, _NW, 8, 128), jnp.float32),
        compiler_params=pltpu.CompilerParams(
            use_tc_tiling_on_sc=False, needs_layout_passes=False),
        scratch_types=[
            pltpu.VMEM((_S, _CH), jnp.int32),
            pltpu.VMEM((2, _CH, _D), jnp.float32),
            pltpu.VMEM((2, _D, 129), jnp.float32),
            pltpu.SemaphoreType.DMA,
            pltpu.SemaphoreType.DMA,
            pltpu.SemaphoreType.DMA,
            pltpu.SemaphoreType.DMA,
        ],
    )
    def body(xk_hbm, table_hbm, out_hbm, kv, rows_v, slab_v, g0, g1, s0, s1):
        gsems = (g0, g1)
        ssems = (s0, s1)
        w = lax.axis_index("s") * 2 + lax.axis_index("c")
        lane16 = lax.iota(jnp.int32, 16)
        dk = [(k * 16 + lane16) * 129 for k in range(4)]

        pltpu.sync_copy(xk_hbm.at[:, w], kv)
        pltpu.async_copy(table_hbm.at[kv.at[0]], rows_v.at[0], gsems[0])

        def out_slab_copy(s, b):
            return [
                pltpu.make_async_copy(
                    slab_v.at[b, pl.ds(dt * 8, 8), pl.ds(0, 128)],
                    out_hbm.at[s, dt, w], ssems[b])
                for dt in range(8)
            ]

        def pair(jj, carry):
            for b in range(2):
                nb = 1 - b
                s = jj * 2 + b

                @pl.when(s + 1 < _NSLAB)
                def _():
                    pltpu.async_copy(
                        table_hbm.at[kv.at[s + 1]], rows_v.at[nb], gsems[nb])

                pltpu.make_async_copy(
                    table_hbm.at[kv.at[s]], rows_v.at[b], gsems[b]).wait()

                @pl.when(s >= 2)
                def _():
                    for c in out_slab_copy(s - 2, b):
                        c.wait()

                flat = slab_v.at[b, 0]
                for j in range(_CH):
                    for k in range(4):
                        vals = rows_v[b, j, pl.ds(k * 16, 16)]
                        plsc.store_scatter(flat, [dk[k] + j], vals)

                for c in out_slab_copy(s, b):
                    c.start()
            return carry

        lax.fori_loop(0, _NSLAB // 2, pair, 0)
        for b in range(2):
            for c in out_slab_copy(_NSLAB - 2 + b, b):
                c.wait()

    return body(xk, table)


def kernel(x, table):
    xk = x.astype(jnp.int32).reshape(_NW, _CH, _S).transpose(2, 0, 1)
    out = _gather(xk, table)
    return out.transpose(2, 4, 0, 1, 3).reshape(_B, _S, _D)


# speedup vs baseline: 1.1645x; 1.1645x over previous
import functools

import jax
import jax.numpy as jnp
from jax import lax
from jax.experimental import pallas as pl
from jax.experimental.pallas import tpu as pltpu
from jax.experimental.pallas import tpu_sc as plsc

_B, _S, _D = 4096, 26, 64
_NW = 32
_CH = 128
_NSLAB = _S


def _gather(xk, table):
    mesh = plsc.VectorSubcoreMesh(core_axis_name="c", subcore_axis_name="s")

    @functools.partial(
        pl.kernel,
        mesh=mesh,
        out_type=jax.ShapeDtypeStruct((_S, 8, _NW, 8, 128), jnp.float32),
        compiler_params=pltpu.CompilerParams(
            use_tc_tiling_on_sc=False, needs_layout_passes=False),
        scratch_types=[
            pltpu.VMEM((_S, _CH), jnp.int32),
            pltpu.VMEM((2, _CH, _D), jnp.float32),
            pltpu.VMEM((2, _D, 129), jnp.float32),
            pltpu.SemaphoreType.DMA,
            pltpu.SemaphoreType.DMA,
            pltpu.SemaphoreType.DMA,
            pltpu.SemaphoreType.DMA,
        ],
    )
    def body(xk_hbm, table_hbm, out_hbm, kv, rows_v, slab_v, g0, g1, s0, s1):
        gsems = (g0, g1)
        ssems = (s0, s1)
        w = lax.axis_index("s") * 2 + lax.axis_index("c")
        lane16 = lax.iota(jnp.int32, 16)
        dk = [k * 16 + lane16 for k in range(4)]

        pltpu.sync_copy(xk_hbm.at[:, w], kv)
        pltpu.async_copy(table_hbm.at[kv.at[0]], rows_v.at[0], gsems[0])

        def out_slab_copy(s, b):
            return [
                pltpu.make_async_copy(
                    slab_v.at[b, pl.ds(dt * 8, 8), pl.ds(0, 128)],
                    out_hbm.at[s, dt, w], ssems[b])
                for dt in range(8)
            ]

        def pair(jj, carry):
            for b in range(2):
                nb = 1 - b
                s = jj * 2 + b

                @pl.when(s + 1 < _NSLAB)
                def _():
                    pltpu.async_copy(
                        table_hbm.at[kv.at[s + 1]], rows_v.at[nb], gsems[nb])

                pltpu.make_async_copy(
                    table_hbm.at[kv.at[s]], rows_v.at[b], gsems[b]).wait()

                @pl.when(s >= 2)
                def _():
                    for c in out_slab_copy(s - 2, b):
                        c.wait()

                def col(i, carry2):
                    j0 = i * 4
                    for jj in range(4):
                        j = j0 + jj
                        jv = jnp.full((16,), j, jnp.int32)
                        for k in range(4):
                            vals = rows_v[b, j, pl.ds(k * 16, 16)]
                            plsc.store_scatter(
                                slab_v.at[b], [dk[k], jv], vals)
                    return carry2

                lax.fori_loop(0, _CH // 4, col, 0)

                for c in out_slab_copy(s, b):
                    c.start()
            return carry

        lax.fori_loop(0, _NSLAB // 2, pair, 0)
        for b in range(2):
            for c in out_slab_copy(_NSLAB - 2 + b, b):
                c.wait()

    return body(xk, table)


def kernel(x, table):
    xk = x.astype(jnp.int32).reshape(_NW, _CH, _S).transpose(2, 0, 1)
    out = _gather(xk, table)
    return out.transpose(2, 4, 0, 1, 3).reshape(_B, _S, _D)
